# baseline (speedup 1.0000x reference)
import functools
import math

import jax
import jax.numpy as jnp
from jax import lax
from jax.experimental import pallas as pl
from jax.experimental.pallas import tpu as pltpu
from jax.experimental.pallas import tpu_sc as plsc

_B = 4
_S = 8192
_D = 1024
_NSEG = 64
_SEGW = _S // _NSEG
_TOPK = 8
_NMACRO = 4
_PAD = 16
_NSRC = 13
_EPS = 1.1920928955078125e-07
_NEG = -3.0e38
_L = 16


def _reduce_body(h_ref, nll_ref, means_ref, lasts_ref, h2_ref, ss_ref):
    x = h_ref[...]
    m = jnp.mean(x, axis=2)
    means_ref[...] = m
    lasts_ref[...] = x[:, :, _SEGW - 1, :]
    nb = m.shape[1]
    h2 = jnp.sum(m * m, axis=2)
    h2_ref[...] = jnp.broadcast_to(h2[:, :, None], (1, nb, _SEGW))
    ss = jnp.mean(nll_ref[...], axis=2)
    ss_ref[...] = jnp.broadcast_to(ss[:, :, None], (1, nb, _SEGW))


def _sqrt16(a):
    i = plsc.bitcast(a, jnp.int32)
    y = plsc.bitcast(jnp.int32(0x5F3759DF) - lax.shift_right_logical(i, 1),
                     jnp.float32)
    for _ in range(5):
        y = y * (1.5 - 0.5 * a * y * y)
    return a * y


def _sc_select_body(nbat, means_hbm, lasts_hbm, h2_hbm, ss_hbm,
                    left_hbm, right_hbm,
                    hbuf, sbuf, idxb, mrows, lrows, sem1, sem2):
    nc = 2
    wid = lax.axis_index("s") * nc + lax.axis_index("c")

    @pl.when(wid < nbat)
    def _():
        b = wid
        iota = lax.iota(jnp.int32, _L)
        zeros_i = jnp.zeros((_L,), jnp.int32)

        pltpu.sync_copy(h2_hbm.at[b], hbuf)
        pltpu.sync_copy(ss_hbm.at[b], sbuf)

        nv = _NSEG // _L
        h = [_sqrt16(plsc.load_gather(hbuf, [iota + k * _L, zeros_i]))
             for k in range(nv)]
        s_sc = [plsc.load_gather(sbuf, [iota + k * _L, zeros_i])
                for k in range(nv)]

        def _zscore(vs):
            tot = vs[0] + vs[1] + vs[2] + vs[3]
            mu = jnp.sum(tot) * (1.0 / _NSEG)
            d = [v - mu for v in vs]
            var = (jnp.sum(d[0] * d[0]) + jnp.sum(d[1] * d[1])
                   + jnp.sum(d[2] * d[2]) + jnp.sum(d[3] * d[3])) * (1.0 / _NSEG)
            sd = _sqrt16(jnp.zeros((_L,), jnp.float32) + var)
            den = jnp.maximum(sd, 1e-6)
            return [dv / den for dv in d]

        hz = _zscore(h)
        sz = _zscore(s_sc)
        w = [hz[k] + sz[k] for k in range(nv)]
        selm = [iota < 0 for _ in range(nv)]

        for _t in range(_TOPK):
            mm = jnp.maximum(jnp.maximum(w[0], w[1]),
                             jnp.maximum(w[2], w[3]))
            m = jnp.max(mm)
            cands = [jnp.where(w[k] >= m, iota + k * _L, _NSEG)
                     for k in range(nv)]
            imin = jnp.min(jnp.minimum(jnp.minimum(cands[0], cands[1]),
                                       jnp.minimum(cands[2], cands[3])))
            for k in range(nv):
                hit = (iota + k * _L) == imin
                selm[k] = selm[k] | hit
                w[k] = jnp.where(hit, _NEG, w[k])

        off = 0
        for k in range(nv):
            sm = selm[k].astype(jnp.int32)
            excl = plsc.cumsum(sm) - sm
            pos = excl + off
            plsc.store_scatter(idxb, [pos], iota + (k * _L + _NSEG * b),
                               mask=selm[k])
            off = off + jnp.sum(sm)

        cp1 = pltpu.async_copy(means_hbm.at[idxb], mrows, sem1)
        cp2 = pltpu.async_copy(lasts_hbm.at[idxb], lrows, sem2)
        cp1.wait()
        cp2.wait()
        pltpu.sync_copy(mrows, left_hbm.at[b])
        pltpu.sync_copy(lrows, right_hbm.at[b])


def _sc_select(meansflat, lastsflat, h2b, ssb):
    f32 = jnp.float32
    nbat = h2b.shape[0]
    mesh = plsc.VectorSubcoreMesh(core_axis_name="c", subcore_axis_name="s")
    fn = functools.partial(
        pl.kernel,
        mesh=mesh,
        compiler_params=pltpu.CompilerParams(needs_layout_passes=False),
        out_type=[jax.ShapeDtypeStruct((nbat, _TOPK, _D), f32),
                  jax.ShapeDtypeStruct((nbat, _TOPK, _D), f32)],
        scratch_types=[
            pltpu.VMEM((_NSEG, _SEGW), f32),
            pltpu.VMEM((_NSEG, _SEGW), f32),
            pltpu.VMEM((_TOPK,), jnp.int32),
            pltpu.VMEM((_TOPK, _D), f32),
            pltpu.VMEM((_TOPK, _D), f32),
            pltpu.SemaphoreType.DMA,
            pltpu.SemaphoreType.DMA,
        ],
    )(functools.partial(_sc_select_body, nbat))
    return fn(meansflat, lastsflat, h2b, ssb)


def _dot(a, b):
    return lax.dot_general(a, b, (((1,), (0,)), ((), ())),
                           preferred_element_type=jnp.float32)


def _dot_t(a, b):
    return lax.dot_general(a, b, (((1,), (1,)), ((), ())),
                           preferred_element_type=jnp.float32)


def _ct(a, b):
    return lax.dot_general(a, b, (((0,), (0,)), ((), ())),
                           preferred_element_type=jnp.float32)


def _finish_body(m0_ref, m1_ref, s0_ref, s1_ref, l8a_ref, l8b_ref,
                 r8a_ref, r8b_ref,
                 q_ref, ws_ref, wk_ref, wv_ref, wo_ref, out_ref):
    f32 = jnp.float32
    gi = lax.broadcasted_iota(jnp.int32, (_NSEG, 8), 0)
    gj = lax.broadcasted_iota(jnp.int32, (_NSEG, 8), 1)
    xm = jnp.where((gj < _NMACRO) & ((gi // 16) == gj), 1.0 / 16.0, 0.0)
    xm = xm + jnp.where(gj == _NMACRO, 1.0 / 64.0, 0.0)
    xl = jnp.where((gj < _NMACRO) & (gi == gj * 16 + 15), 1.0, 0.0)
    xl = xl + jnp.where((gj == _NMACRO) & (gi == _NSEG - 1), 1.0, 0.0)

    left_parts = []
    right_parts = []
    for b in range(_B):
        l8 = l8a_ref[b] if b < 2 else l8b_ref[b - 2]
        r8 = r8a_ref[b] if b < 2 else r8b_ref[b - 2]
        mns = m0_ref[b] if b < 2 else m1_ref[b - 2]
        lst = s0_ref[b] if b < 2 else s1_ref[b - 2]
        left_parts.append(jnp.concatenate(
            [l8, _ct(xm, mns)], axis=0))
        right_parts.append(jnp.concatenate(
            [r8, _ct(xl, lst)], axis=0))
    left = jnp.concatenate(left_parts, axis=0)
    right = jnp.concatenate(right_parts, axis=0)

    ws = ws_ref[...]
    summ = _dot_t(left, ws[:, :_D]) + _dot_t(right, ws[:, _D:])
    ms = jnp.mean(summ * summ, axis=1, keepdims=True)
    sources = summ * lax.rsqrt(ms + _EPS)

    keys = _dot_t(sources, wk_ref[...])
    vals = _dot_t(sources, wv_ref[...])
    q = q_ref[...]
    att = _dot_t(q, keys) / math.sqrt(_D)
    cols = lax.broadcasted_iota(jnp.int32, (64, _PAD), 1)
    pad_mask = cols >= _NSRC
    prefix_parts = []
    for b in range(_B):
        a_b = jnp.where(pad_mask, _NEG, att[:, b * _PAD:(b + 1) * _PAD])
        a_b = a_b - jnp.max(a_b, axis=1, keepdims=True)
        e = jnp.exp(a_b)
        p_b = e / jnp.sum(e, axis=1, keepdims=True)
        prefix_parts.append(_dot(p_b, vals[b * _PAD:(b + 1) * _PAD]))
    prefix = jnp.concatenate(prefix_parts, axis=0)
    out = _dot_t(prefix, wo_ref[...])
    out_ref[...] = out.reshape(_B, 64, _D)


def kernel(prev_hidden, prev_nll, query, W_sum, W_k, W_v, W_o):
    f32 = jnp.float32
    h4 = prev_hidden.reshape(_B, _NSEG, _SEGW, _D)

    nll3 = prev_nll.reshape(_B, _NSEG, _SEGW)
    nb = 16
    hb = _B // 2

    def _half(h4h, nll3h):
        return pl.pallas_call(
            _reduce_body,
            grid=(hb, _NSEG // nb),
            in_specs=[
                pl.BlockSpec((1, nb, _SEGW, _D), lambda b, n: (b, n, 0, 0)),
                pl.BlockSpec((1, nb, _SEGW), lambda b, n: (b, n, 0))],
            out_specs=[pl.BlockSpec((1, nb, _D), lambda b, n: (b, n, 0)),
                       pl.BlockSpec((1, nb, _D), lambda b, n: (b, n, 0)),
                       pl.BlockSpec((1, nb, _SEGW), lambda b, n: (b, n, 0)),
                       pl.BlockSpec((1, nb, _SEGW), lambda b, n: (b, n, 0))],
            out_shape=[jax.ShapeDtypeStruct((hb, _NSEG, _D), f32),
                       jax.ShapeDtypeStruct((hb, _NSEG, _D), f32),
                       jax.ShapeDtypeStruct((hb, _NSEG, _SEGW), f32),
                       jax.ShapeDtypeStruct((hb, _NSEG, _SEGW), f32)],
        )(h4h, nll3h)

    means0, lasts0, h2b0, ssb0 = _half(h4[:hb], nll3[:hb])
    left8a, right8a = _sc_select(means0.reshape(hb * _NSEG, _D),
                                 lasts0.reshape(hb * _NSEG, _D), h2b0, ssb0)
    means1, lasts1, h2b1, ssb1 = _half(h4[hb:], nll3[hb:])
    left8b, right8b = _sc_select(means1.reshape(hb * _NSEG, _D),
                                 lasts1.reshape(hb * _NSEG, _D), h2b1, ssb1)
    out = pl.pallas_call(
        _finish_body,
        in_specs=[
            pl.BlockSpec((hb, _NSEG, _D), lambda: (0, 0, 0)),
            pl.BlockSpec((hb, _NSEG, _D), lambda: (0, 0, 0)),
            pl.BlockSpec((hb, _NSEG, _D), lambda: (0, 0, 0)),
            pl.BlockSpec((hb, _NSEG, _D), lambda: (0, 0, 0)),
            pl.BlockSpec((hb, _TOPK, _D), lambda: (0, 0, 0)),
            pl.BlockSpec((hb, _TOPK, _D), lambda: (0, 0, 0)),
            pl.BlockSpec((hb, _TOPK, _D), lambda: (0, 0, 0)),
            pl.BlockSpec((hb, _TOPK, _D), lambda: (0, 0, 0)),
            pl.BlockSpec((64, _D), lambda: (0, 0)),
            pl.BlockSpec((_D, 2 * _D), lambda: (0, 0)),
            pl.BlockSpec((_D, _D), lambda: (0, 0)),
            pl.BlockSpec((_D, _D), lambda: (0, 0)),
            pl.BlockSpec((_D, _D), lambda: (0, 0)),
        ],
        out_specs=pl.BlockSpec((_B, 64, _D), lambda: (0, 0, 0)),
        out_shape=jax.ShapeDtypeStruct((_B, 64, _D), f32),
    )(means0, means1, lasts0, lasts1, left8a, left8b, right8a, right8b,
      query, W_sum, W_k, W_v, W_o)
    return out


# speedup vs baseline: 2.0494x; 2.0494x over previous
import functools
import math

import jax
import jax.numpy as jnp
from jax import lax
from jax.experimental import pallas as pl
from jax.experimental.pallas import tpu as pltpu
from jax.experimental.pallas import tpu_sc as plsc

_B = 4
_S = 8192
_D = 1024
_NSEG = 64
_SEGW = _S // _NSEG
_TOPK = 8
_NMACRO = 4
_PAD = 16
_NSRC = 13
_EPS = 1.1920928955078125e-07
_NEG = -3.0e38
_L = 16


def _reduce_body(h_ref, nll_ref, means_ref, lasts_ref, h2_ref, ss_ref):
    x = h_ref[...]
    m = jnp.mean(x, axis=2)
    means_ref[...] = m
    lasts_ref[...] = x[:, :, _SEGW - 1, :]
    nb = m.shape[1]
    h2 = jnp.sum(m * m, axis=2)
    h2_ref[...] = jnp.broadcast_to(h2[:, :, None], (1, nb, _SEGW))
    ss = jnp.mean(nll_ref[...], axis=2)
    ss_ref[...] = jnp.broadcast_to(ss[:, :, None], (1, nb, _SEGW))


def _sqrt16(a):
    i = plsc.bitcast(a, jnp.int32)
    y = plsc.bitcast(jnp.int32(0x5F3759DF) - lax.shift_right_logical(i, 1),
                     jnp.float32)
    for _ in range(5):
        y = y * (1.5 - 0.5 * a * y * y)
    return a * y


def _sc_select_body(nbat, means_hbm, lasts_hbm, h2_hbm, ss_hbm,
                    left_hbm, right_hbm,
                    hbuf, sbuf, idxb, mrows, lrows, sem1, sem2):
    nc = 2
    wid = lax.axis_index("s") * nc + lax.axis_index("c")

    @pl.when(wid < nbat)
    def _():
        b = wid
        iota = lax.iota(jnp.int32, _L)
        zeros_i = jnp.zeros((_L,), jnp.int32)

        pltpu.sync_copy(h2_hbm.at[b], hbuf)
        pltpu.sync_copy(ss_hbm.at[b], sbuf)

        nv = _NSEG // _L
        h = [_sqrt16(plsc.load_gather(hbuf, [iota + k * _L, zeros_i]))
             for k in range(nv)]
        s_sc = [plsc.load_gather(sbuf, [iota + k * _L, zeros_i])
                for k in range(nv)]

        def _zscore(vs):
            tot = vs[0] + vs[1] + vs[2] + vs[3]
            mu = jnp.sum(tot) * (1.0 / _NSEG)
            d = [v - mu for v in vs]
            var = (jnp.sum(d[0] * d[0]) + jnp.sum(d[1] * d[1])
                   + jnp.sum(d[2] * d[2]) + jnp.sum(d[3] * d[3])) * (1.0 / _NSEG)
            sd = _sqrt16(jnp.zeros((_L,), jnp.float32) + var)
            den = jnp.maximum(sd, 1e-6)
            return [dv / den for dv in d]

        hz = _zscore(h)
        sz = _zscore(s_sc)
        w = [hz[k] + sz[k] for k in range(nv)]
        selm = [iota < 0 for _ in range(nv)]

        for _t in range(_TOPK):
            mm = jnp.maximum(jnp.maximum(w[0], w[1]),
                             jnp.maximum(w[2], w[3]))
            m = jnp.max(mm)
            cands = [jnp.where(w[k] >= m, iota + k * _L, _NSEG)
                     for k in range(nv)]
            imin = jnp.min(jnp.minimum(jnp.minimum(cands[0], cands[1]),
                                       jnp.minimum(cands[2], cands[3])))
            for k in range(nv):
                hit = (iota + k * _L) == imin
                selm[k] = selm[k] | hit
                w[k] = jnp.where(hit, _NEG, w[k])

        off = 0
        for k in range(nv):
            sm = selm[k].astype(jnp.int32)
            excl = plsc.cumsum(sm) - sm
            pos = excl + off
            plsc.store_scatter(idxb, [pos], iota + (k * _L + _NSEG * b),
                               mask=selm[k])
            off = off + jnp.sum(sm)

        cp1 = pltpu.async_copy(means_hbm.at[idxb], mrows, sem1)
        cp2 = pltpu.async_copy(lasts_hbm.at[idxb], lrows, sem2)
        cp1.wait()
        cp2.wait()
        pltpu.sync_copy(mrows, left_hbm.at[b])
        pltpu.sync_copy(lrows, right_hbm.at[b])


def _sc_select(meansflat, lastsflat, h2b, ssb):
    f32 = jnp.float32
    nbat = h2b.shape[0]
    mesh = plsc.VectorSubcoreMesh(core_axis_name="c", subcore_axis_name="s")
    fn = functools.partial(
        pl.kernel,
        mesh=mesh,
        compiler_params=pltpu.CompilerParams(needs_layout_passes=False),
        out_type=[jax.ShapeDtypeStruct((nbat, _TOPK, _D), f32),
                  jax.ShapeDtypeStruct((nbat, _TOPK, _D), f32)],
        scratch_types=[
            pltpu.VMEM((_NSEG, _SEGW), f32),
            pltpu.VMEM((_NSEG, _SEGW), f32),
            pltpu.VMEM((_TOPK,), jnp.int32),
            pltpu.VMEM((_TOPK, _D), f32),
            pltpu.VMEM((_TOPK, _D), f32),
            pltpu.SemaphoreType.DMA,
            pltpu.SemaphoreType.DMA,
        ],
    )(functools.partial(_sc_select_body, nbat))
    return fn(meansflat, lastsflat, h2b, ssb)


def _dot(a, b):
    return lax.dot_general(a, b, (((1,), (0,)), ((), ())),
                           preferred_element_type=jnp.float32)


def _dot_t(a, b):
    return lax.dot_general(a, b, (((1,), (1,)), ((), ())),
                           preferred_element_type=jnp.float32)


def _ct(a, b):
    return lax.dot_general(a, b, (((0,), (0,)), ((), ())),
                           preferred_element_type=jnp.float32)


def _finish_body(m0_ref, m1_ref, s0_ref, s1_ref, l8a_ref, l8b_ref,
                 r8a_ref, r8b_ref,
                 q_ref, ws_ref, wk_ref, wv_ref, wo_ref, out_ref):
    f32 = jnp.float32
    gi = lax.broadcasted_iota(jnp.int32, (_NSEG, 8), 0)
    gj = lax.broadcasted_iota(jnp.int32, (_NSEG, 8), 1)
    xm = jnp.where((gj < _NMACRO) & ((gi // 16) == gj), 1.0 / 16.0, 0.0)
    xm = xm + jnp.where(gj == _NMACRO, 1.0 / 64.0, 0.0)
    xl = jnp.where((gj < _NMACRO) & (gi == gj * 16 + 15), 1.0, 0.0)
    xl = xl + jnp.where((gj == _NMACRO) & (gi == _NSEG - 1), 1.0, 0.0)

    left_parts = []
    right_parts = []
    for b in range(_B):
        l8 = l8a_ref[b] if b < 2 else l8b_ref[b - 2]
        r8 = r8a_ref[b] if b < 2 else r8b_ref[b - 2]
        mns = m0_ref[b] if b < 2 else m1_ref[b - 2]
        lst = s0_ref[b] if b < 2 else s1_ref[b - 2]
        left_parts.append(jnp.concatenate(
            [l8, _ct(xm, mns)], axis=0))
        right_parts.append(jnp.concatenate(
            [r8, _ct(xl, lst)], axis=0))
    left = jnp.concatenate(left_parts, axis=0)
    right = jnp.concatenate(right_parts, axis=0)

    ws = ws_ref[...]
    summ = _dot_t(left, ws[:, :_D]) + _dot_t(right, ws[:, _D:])
    ms = jnp.mean(summ * summ, axis=1, keepdims=True)
    sources = summ * lax.rsqrt(ms + _EPS)

    keys = _dot_t(sources, wk_ref[...])
    vals = _dot_t(sources, wv_ref[...])
    q = q_ref[...]
    att = _dot_t(q, keys) / math.sqrt(_D)
    cols = lax.broadcasted_iota(jnp.int32, (64, _PAD), 1)
    pad_mask = cols >= _NSRC
    prefix_parts = []
    for b in range(_B):
        a_b = jnp.where(pad_mask, _NEG, att[:, b * _PAD:(b + 1) * _PAD])
        a_b = a_b - jnp.max(a_b, axis=1, keepdims=True)
        e = jnp.exp(a_b)
        p_b = e / jnp.sum(e, axis=1, keepdims=True)
        prefix_parts.append(_dot(p_b, vals[b * _PAD:(b + 1) * _PAD]))
    prefix = jnp.concatenate(prefix_parts, axis=0)
    out = _dot_t(prefix, wo_ref[...])
    out_ref[...] = out.reshape(_B, 64, _D)


def kernel(prev_hidden, prev_nll, query, W_sum, W_k, W_v, W_o):
    f32 = jnp.float32
    h4 = prev_hidden.reshape(_B, _NSEG, _SEGW, _D)

    nll3 = prev_nll.reshape(_B, _NSEG, _SEGW)
    nb = 16
    hb = _B // 2

    def _half(base):
        return pl.pallas_call(
            _reduce_body,
            grid=(hb, _NSEG // nb),
            in_specs=[
                pl.BlockSpec((1, nb, _SEGW, _D),
                             lambda b, n: (b + base, n, 0, 0)),
                pl.BlockSpec((1, nb, _SEGW), lambda b, n: (b + base, n, 0))],
            out_specs=[pl.BlockSpec((1, nb, _D), lambda b, n: (b, n, 0)),
                       pl.BlockSpec((1, nb, _D), lambda b, n: (b, n, 0)),
                       pl.BlockSpec((1, nb, _SEGW), lambda b, n: (b, n, 0)),
                       pl.BlockSpec((1, nb, _SEGW), lambda b, n: (b, n, 0))],
            out_shape=[jax.ShapeDtypeStruct((hb, _NSEG, _D), f32),
                       jax.ShapeDtypeStruct((hb, _NSEG, _D), f32),
                       jax.ShapeDtypeStruct((hb, _NSEG, _SEGW), f32),
                       jax.ShapeDtypeStruct((hb, _NSEG, _SEGW), f32)],
        )(h4, nll3)

    means0, lasts0, h2b0, ssb0 = _half(0)
    left8a, right8a = _sc_select(means0.reshape(hb * _NSEG, _D),
                                 lasts0.reshape(hb * _NSEG, _D), h2b0, ssb0)
    means1, lasts1, h2b1, ssb1 = _half(hb)
    left8b, right8b = _sc_select(means1.reshape(hb * _NSEG, _D),
                                 lasts1.reshape(hb * _NSEG, _D), h2b1, ssb1)
    out = pl.pallas_call(
        _finish_body,
        in_specs=[
            pl.BlockSpec((hb, _NSEG, _D), lambda: (0, 0, 0)),
            pl.BlockSpec((hb, _NSEG, _D), lambda: (0, 0, 0)),
            pl.BlockSpec((hb, _NSEG, _D), lambda: (0, 0, 0)),
            pl.BlockSpec((hb, _NSEG, _D), lambda: (0, 0, 0)),
            pl.BlockSpec((hb, _TOPK, _D), lambda: (0, 0, 0)),
            pl.BlockSpec((hb, _TOPK, _D), lambda: (0, 0, 0)),
            pl.BlockSpec((hb, _TOPK, _D), lambda: (0, 0, 0)),
            pl.BlockSpec((hb, _TOPK, _D), lambda: (0, 0, 0)),
            pl.BlockSpec((64, _D), lambda: (0, 0)),
            pl.BlockSpec((_D, 2 * _D), lambda: (0, 0)),
            pl.BlockSpec((_D, _D), lambda: (0, 0)),
            pl.BlockSpec((_D, _D), lambda: (0, 0)),
            pl.BlockSpec((_D, _D), lambda: (0, 0)),
        ],
        out_specs=pl.BlockSpec((_B, 64, _D), lambda: (0, 0, 0)),
        out_shape=jax.ShapeDtypeStruct((_B, 64, _D), f32),
    )(means0, means1, lasts0, lasts1, left8a, left8b, right8a, right8b,
      query, W_sum, W_k, W_v, W_o)
    return out


# speedup vs baseline: 2.1587x; 1.0533x over previous
import functools
import math

import jax
import jax.numpy as jnp
from jax import lax
from jax.experimental import pallas as pl
from jax.experimental.pallas import tpu as pltpu
from jax.experimental.pallas import tpu_sc as plsc

_B = 4
_S = 8192
_D = 1024
_NSEG = 64
_SEGW = _S // _NSEG
_TOPK = 8
_NMACRO = 4
_PAD = 16
_NSRC = 13
_EPS = 1.1920928955078125e-07
_NEG = -3.0e38
_L = 16


def _reduce_body(h_ref, nll_ref, means_ref, lasts_ref, h2_ref, ss_ref):
    x = h_ref[...]
    m = jnp.mean(x, axis=2)
    means_ref[...] = m
    lasts_ref[...] = x[:, :, _SEGW - 1, :]
    nb = m.shape[1]
    h2 = jnp.sum(m * m, axis=2)
    h2_ref[...] = jnp.broadcast_to(h2[:, :, None], (1, nb, _SEGW))
    ss = jnp.mean(nll_ref[...], axis=2)
    ss_ref[...] = jnp.broadcast_to(ss[:, :, None], (1, nb, _SEGW))


def _sqrt16(a):
    i = plsc.bitcast(a, jnp.int32)
    y = plsc.bitcast(jnp.int32(0x5F3759DF) - lax.shift_right_logical(i, 1),
                     jnp.float32)
    for _ in range(5):
        y = y * (1.5 - 0.5 * a * y * y)
    return a * y


def _sc_select_body(means_hbm, lasts_hbm, h2_hbm, ss_hbm, left_hbm, right_hbm,
                    hbuf, sbuf, idxb, mrows, lrows, sem1, sem2):
    wid = lax.axis_index("s") + lax.axis_index("c")

    @pl.when(wid < _B)
    def _():
        b = wid
        iota = lax.iota(jnp.int32, _L)
        zeros_i = jnp.zeros((_L,), jnp.int32)

        pltpu.sync_copy(h2_hbm.at[b], hbuf)
        pltpu.sync_copy(ss_hbm.at[b], sbuf)

        nv = _NSEG // _L
        h = [_sqrt16(plsc.load_gather(hbuf, [iota + k * _L, zeros_i]))
             for k in range(nv)]
        s_sc = [plsc.load_gather(sbuf, [iota + k * _L, zeros_i])
                for k in range(nv)]

        def _zscore(vs):
            tot = vs[0] + vs[1] + vs[2] + vs[3]
            mu = jnp.sum(tot) * (1.0 / _NSEG)
            d = [v - mu for v in vs]
            var = (jnp.sum(d[0] * d[0]) + jnp.sum(d[1] * d[1])
                   + jnp.sum(d[2] * d[2]) + jnp.sum(d[3] * d[3])) * (1.0 / _NSEG)
            sd = _sqrt16(jnp.zeros((_L,), jnp.float32) + var)
            den = jnp.maximum(sd, 1e-6)
            return [dv / den for dv in d]

        hz = _zscore(h)
        sz = _zscore(s_sc)
        w = [hz[k] + sz[k] for k in range(nv)]
        selm = [iota < 0 for _ in range(nv)]

        for _t in range(_TOPK):
            mm = jnp.maximum(jnp.maximum(w[0], w[1]),
                             jnp.maximum(w[2], w[3]))
            m = jnp.max(mm)
            cands = [jnp.where(w[k] >= m, iota + k * _L, _NSEG)
                     for k in range(nv)]
            imin = jnp.min(jnp.minimum(jnp.minimum(cands[0], cands[1]),
                                       jnp.minimum(cands[2], cands[3])))
            for k in range(nv):
                hit = (iota + k * _L) == imin
                selm[k] = selm[k] | hit
                w[k] = jnp.where(hit, _NEG, w[k])

        off = 0
        for k in range(nv):
            sm = selm[k].astype(jnp.int32)
            excl = plsc.cumsum(sm) - sm
            pos = excl + off
            plsc.store_scatter(idxb, [pos], iota + (k * _L + _NSEG * b),
                               mask=selm[k])
            off = off + jnp.sum(sm)

        cp1 = pltpu.async_copy(means_hbm.at[idxb], mrows, sem1)
        cp2 = pltpu.async_copy(lasts_hbm.at[idxb], lrows, sem2)
        cp1.wait()
        cp2.wait()
        pltpu.sync_copy(mrows, left_hbm.at[b])
        pltpu.sync_copy(lrows, right_hbm.at[b])


def _sc_select(meansflat, lastsflat, h2b, ssb):
    f32 = jnp.float32
    mesh = plsc.VectorSubcoreMesh(core_axis_name="c", subcore_axis_name="s", num_cores=1)
    fn = functools.partial(
        pl.kernel,
        mesh=mesh,
        compiler_params=pltpu.CompilerParams(needs_layout_passes=False),
        out_type=[jax.ShapeDtypeStruct((_B, _TOPK, _D), f32),
                  jax.ShapeDtypeStruct((_B, _TOPK, _D), f32)],
        scratch_types=[
            pltpu.VMEM((_NSEG, _SEGW), f32),
            pltpu.VMEM((_NSEG, _SEGW), f32),
            pltpu.VMEM((_TOPK,), jnp.int32),
            pltpu.VMEM((_TOPK, _D), f32),
            pltpu.VMEM((_TOPK, _D), f32),
            pltpu.SemaphoreType.DMA,
            pltpu.SemaphoreType.DMA,
        ],
    )(_sc_select_body)
    return fn(meansflat, lastsflat, h2b, ssb)


def _dot(a, b):
    return lax.dot_general(a, b, (((1,), (0,)), ((), ())),
                           preferred_element_type=jnp.float32)


def _dot_t(a, b):
    return lax.dot_general(a, b, (((1,), (1,)), ((), ())),
                           preferred_element_type=jnp.float32)


def _ct(a, b):
    return lax.dot_general(a, b, (((0,), (0,)), ((), ())),
                           preferred_element_type=jnp.float32)


def _finish_body(means_ref, lasts_ref, l8_ref, r8_ref, q_ref,
                 ws_ref, wk_ref, wv_ref, wo_ref, out_ref):
    f32 = jnp.float32
    gi = lax.broadcasted_iota(jnp.int32, (_NSEG, 8), 0)
    gj = lax.broadcasted_iota(jnp.int32, (_NSEG, 8), 1)
    xm = jnp.where((gj < _NMACRO) & ((gi // 16) == gj), 1.0 / 16.0, 0.0)
    xm = xm + jnp.where(gj == _NMACRO, 1.0 / 64.0, 0.0)
    xl = jnp.where((gj < _NMACRO) & (gi == gj * 16 + 15), 1.0, 0.0)
    xl = xl + jnp.where((gj == _NMACRO) & (gi == _NSEG - 1), 1.0, 0.0)

    left_parts = []
    right_parts = []
    for b in range(_B):
        left_parts.append(jnp.concatenate(
            [l8_ref[b], _ct(xm, means_ref[b])], axis=0))
        right_parts.append(jnp.concatenate(
            [r8_ref[b], _ct(xl, lasts_ref[b])], axis=0))
    left = jnp.concatenate(left_parts, axis=0)
    right = jnp.concatenate(right_parts, axis=0)

    ws = ws_ref[...]
    summ = _dot_t(left, ws[:, :_D]) + _dot_t(right, ws[:, _D:])
    ms = jnp.mean(summ * summ, axis=1, keepdims=True)
    sources = summ * lax.rsqrt(ms + _EPS)

    keys = _dot_t(sources, wk_ref[...])
    vals = _dot_t(sources, wv_ref[...])
    q = q_ref[...]
    att = _dot_t(q, keys) / math.sqrt(_D)
    cols = lax.broadcasted_iota(jnp.int32, (64, _PAD), 1)
    pad_mask = cols >= _NSRC
    prefix_parts = []
    for b in range(_B):
        a_b = jnp.where(pad_mask, _NEG, att[:, b * _PAD:(b + 1) * _PAD])
        a_b = a_b - jnp.max(a_b, axis=1, keepdims=True)
        e = jnp.exp(a_b)
        p_b = e / jnp.sum(e, axis=1, keepdims=True)
        prefix_parts.append(_dot(p_b, vals[b * _PAD:(b + 1) * _PAD]))
    prefix = jnp.concatenate(prefix_parts, axis=0)
    out = _dot_t(prefix, wo_ref[...])
    out_ref[...] = out.reshape(_B, 64, _D)


def kernel(prev_hidden, prev_nll, query, W_sum, W_k, W_v, W_o):
    f32 = jnp.float32
    h4 = prev_hidden.reshape(_B, _NSEG, _SEGW, _D)

    nll3 = prev_nll.reshape(_B, _NSEG, _SEGW)
    nb = 16
    means, lasts, h2b, ssb = pl.pallas_call(
        _reduce_body,
        grid=(_B, _NSEG // nb),
        in_specs=[pl.BlockSpec((1, nb, _SEGW, _D), lambda b, n: (b, n, 0, 0)),
                  pl.BlockSpec((1, nb, _SEGW), lambda b, n: (b, n, 0))],
        out_specs=[pl.BlockSpec((1, nb, _D), lambda b, n: (b, n, 0)),
                   pl.BlockSpec((1, nb, _D), lambda b, n: (b, n, 0)),
                   pl.BlockSpec((1, nb, _SEGW), lambda b, n: (b, n, 0)),
                   pl.BlockSpec((1, nb, _SEGW), lambda b, n: (b, n, 0))],
        out_shape=[jax.ShapeDtypeStruct((_B, _NSEG, _D), f32),
                   jax.ShapeDtypeStruct((_B, _NSEG, _D), f32),
                   jax.ShapeDtypeStruct((_B, _NSEG, _SEGW), f32),
                   jax.ShapeDtypeStruct((_B, _NSEG, _SEGW), f32)],
    )(h4, nll3)

    left8, right8 = _sc_select(means.reshape(_B * _NSEG, _D),
                               lasts.reshape(_B * _NSEG, _D), h2b, ssb)

    out = pl.pallas_call(
        _finish_body,
        in_specs=[
            pl.BlockSpec((_B, _NSEG, _D), lambda: (0, 0, 0)),
            pl.BlockSpec((_B, _NSEG, _D), lambda: (0, 0, 0)),
            pl.BlockSpec((_B, _TOPK, _D), lambda: (0, 0, 0)),
            pl.BlockSpec((_B, _TOPK, _D), lambda: (0, 0, 0)),
            pl.BlockSpec((64, _D), lambda: (0, 0)),
            pl.BlockSpec((_D, 2 * _D), lambda: (0, 0)),
            pl.BlockSpec((_D, _D), lambda: (0, 0)),
            pl.BlockSpec((_D, _D), lambda: (0, 0)),
            pl.BlockSpec((_D, _D), lambda: (0, 0)),
        ],
        out_specs=pl.BlockSpec((_B, 64, _D), lambda: (0, 0, 0)),
        out_shape=jax.ShapeDtypeStruct((_B, 64, _D), f32),
    )(means, lasts, left8, right8, query, W_sum, W_k, W_v, W_o)
    return out
